# Initial kernel scaffold; baseline (speedup 1.0000x reference)
import jax
import jax.numpy as jnp
from jax.experimental import pallas as pl


def kernel(features, conv_w, conv_b, cls_w, cls_b, bbox_w, bbox_b):
    raise NotImplementedError("write your pallas kernel here")


# speedup vs baseline: 11.9759x; 11.9759x over previous
import numpy as np
import jax
import jax.numpy as jnp
from jax.experimental import pallas as pl
from jax.experimental.pallas import tpu as pltpu

B, C, H, W = 2, 256, 50, 50
A = 3
STRIDE = 16
IMG = float(H * STRIDE)
PRE_NMS = 2000
POST_NMS = 1000
NMS_THRESH = 0.7
BBOX_CLIP = float(np.log(1000.0 / 16.0))
NPAD = 2048
BLK = 128
NBLK = NPAD // BLK


def _make_anchors():
    size = 128.0
    ratios = np.array([0.5, 1.0, 2.0], dtype=np.float64)
    h_r = np.sqrt(ratios)
    w_r = 1.0 / h_r
    ws = (w_r * size) / 2.0
    hs = (h_r * size) / 2.0
    base = np.stack([-ws, -hs, ws, hs], axis=1)
    sx = (np.arange(W, dtype=np.float64) + 0.5) * STRIDE
    sy = (np.arange(H, dtype=np.float64) + 0.5) * STRIDE
    yy, xx = np.meshgrid(sy, sx, indexing='ij')
    shifts = np.stack([xx.ravel(), yy.ravel(), xx.ravel(), yy.ravel()], axis=1)
    anchors = (shifts[:, None, :] + base[None, :, :]).reshape(-1, 4)
    return jnp.asarray(anchors, dtype=jnp.float32)


_ANCHORS = _make_anchors()


CH = 8
NCH = BLK // CH


def _nms_kernel(boxes8_ref, bt_ref, keep_ref):
    bt = bt_ref[0]
    x0r = bt[0:1]
    y0r = bt[1:2]
    x1r = bt[2:3]
    y1r = bt[3:4]
    area_r = (x1r - x0r) * (y1r - y0r)

    col128 = jax.lax.broadcasted_iota(jnp.int32, (1, BLK), 1)
    colg = jax.lax.broadcasted_iota(jnp.int32, (1, NPAD), 1)
    sub8 = jax.lax.broadcasted_iota(jnp.int32, (CH, 1), 0)
    keep = jnp.ones((1, NPAD), dtype=jnp.float32)

    for b in range(NBLK):
        lo = b * BLK
        kb0 = keep[:, lo:lo + BLK]
        acc0 = jnp.zeros((1, NPAD), dtype=jnp.float32)

        def chunk_body(k, carry, b=b, lo=lo):
            kb, acc = carry
            ch = boxes8_ref[0, b * NCH + k]
            x0c = ch[:, 0:1]
            y0c = ch[:, 1:2]
            x1c = ch[:, 2:3]
            y1c = ch[:, 3:4]
            area_c = (x1c - x0c) * (y1c - y0c)
            iw = jnp.maximum(jnp.minimum(x1c, x1r) - jnp.maximum(x0c, x0r), 0.0)
            ih = jnp.maximum(jnp.minimum(y1c, y1r) - jnp.maximum(y0c, y0r), 0.0)
            inter = iw * ih
            union = area_c + area_r - inter
            iou = inter / jnp.maximum(union, 1e-9)
            sup8 = jnp.where(iou > NMS_THRESH, 1.0, 0.0)
            kcol = jnp.zeros((CH, 1), dtype=jnp.float32)
            for j in range(CH):
                ii = k * CH + j
                k_i = jnp.sum(jnp.where(col128 == ii, kb, 0.0))
                row = sup8[j:j + 1, lo:lo + BLK]
                hit = jnp.where((col128 > ii) & (row > 0.0), 1.0, 0.0)
                kb = kb * (1.0 - k_i * hit)
                kcol = jnp.where(sub8 == j, k_i, kcol)
            acc = jnp.maximum(acc, jnp.max(sup8 * kcol, axis=0, keepdims=True))
            return kb, acc

        kb, acc = jax.lax.fori_loop(0, NCH, chunk_body, (kb0, acc0))
        pieces = []
        if lo > 0:
            pieces.append(keep[:, :lo])
        pieces.append(kb)
        if lo + BLK < NPAD:
            pieces.append(keep[:, lo + BLK:])
        keep = jnp.concatenate(pieces, axis=1) if len(pieces) > 1 else kb
        keep = keep * jnp.where((colg >= lo + BLK) & (acc > 0.0), 0.0, 1.0)

    keep_ref[0] = keep


def _run_nms(top_boxes):
    pad = jnp.zeros((B, NPAD - PRE_NMS, 4), dtype=jnp.float32)
    bx = jnp.concatenate([top_boxes, pad], axis=1)
    bt = jnp.transpose(bx, (0, 2, 1))
    bx8 = bx.reshape(B, NPAD // CH, CH, 4)
    keep = pl.pallas_call(
        _nms_kernel,
        out_shape=jax.ShapeDtypeStruct((B, 1, NPAD), jnp.float32),
        grid=(B,),
        in_specs=[
            pl.BlockSpec((1, NPAD // CH, CH, 4), lambda b: (b, 0, 0, 0)),
            pl.BlockSpec((1, 4, NPAD), lambda b: (b, 0, 0)),
        ],
        out_specs=pl.BlockSpec((1, 1, NPAD), lambda b: (b, 0, 0)),
    )(bx8, bt)
    return keep[:, 0, :PRE_NMS]


def _conv2d(x, w, b, pad):
    out = jax.lax.conv_general_dilated(
        x, w, (1, 1), [(pad, pad), (pad, pad)],
        dimension_numbers=('NCHW', 'OIHW', 'NCHW'))
    return out + b[None, :, None, None]


def _decode(deltas, anchors):
    wa = anchors[:, 2] - anchors[:, 0]
    ha = anchors[:, 3] - anchors[:, 1]
    cxa = anchors[:, 0] + 0.5 * wa
    cya = anchors[:, 1] + 0.5 * ha
    dx = deltas[..., 0]
    dy = deltas[..., 1]
    dw = jnp.minimum(deltas[..., 2], BBOX_CLIP)
    dh = jnp.minimum(deltas[..., 3], BBOX_CLIP)
    cx = dx * wa + cxa
    cy = dy * ha + cya
    w = wa * jnp.exp(dw)
    h = ha * jnp.exp(dh)
    boxes = jnp.stack([cx - 0.5 * w, cy - 0.5 * h, cx + 0.5 * w, cy + 0.5 * h],
                      axis=-1)
    return jnp.clip(boxes, 0.0, IMG)


def kernel(features, conv_w, conv_b, cls_w, cls_b, bbox_w, bbox_b):
    t = jax.nn.relu(_conv2d(features, conv_w, conv_b, 1))
    logits = _conv2d(t, cls_w, cls_b, 0)
    deltas = _conv2d(t, bbox_w, bbox_b, 0)
    logits = jnp.transpose(logits, (0, 2, 3, 1)).reshape(B, -1)
    deltas = deltas.reshape(B, A, 4, H, W)
    deltas = jnp.transpose(deltas, (0, 3, 4, 1, 2)).reshape(B, -1, 4)
    scores = jax.nn.sigmoid(logits)
    boxes = _decode(deltas, _ANCHORS)

    top_scores, idx = jax.lax.top_k(scores, PRE_NMS)
    top_boxes = jnp.take_along_axis(boxes, idx[..., None], axis=1)

    keep = _run_nms(top_boxes)

    sel = jnp.where(keep > 0.0, top_scores, -jnp.inf)
    out_scores, oidx = jax.lax.top_k(sel, POST_NMS)
    out_boxes = jnp.take_along_axis(top_boxes, oidx[..., None], axis=1)
    out_scores = jnp.where(jnp.isfinite(out_scores), out_scores, 0.0)
    return jnp.concatenate([out_boxes, out_scores[..., None]], axis=-1)


# speedup vs baseline: 11.9760x; 1.0000x over previous
import numpy as np
import jax
import jax.numpy as jnp
from jax.experimental import pallas as pl
from jax.experimental.pallas import tpu as pltpu

B, C, H, W = 2, 256, 50, 50
A = 3
STRIDE = 16
IMG = float(H * STRIDE)
PRE_NMS = 2000
POST_NMS = 1000
NMS_THRESH = 0.7
BBOX_CLIP = float(np.log(1000.0 / 16.0))
NPAD = 2048
BLK = 128
NBLK = NPAD // BLK


def _make_anchors():
    size = 128.0
    ratios = np.array([0.5, 1.0, 2.0], dtype=np.float64)
    h_r = np.sqrt(ratios)
    w_r = 1.0 / h_r
    ws = (w_r * size) / 2.0
    hs = (h_r * size) / 2.0
    base = np.stack([-ws, -hs, ws, hs], axis=1)
    sx = (np.arange(W, dtype=np.float64) + 0.5) * STRIDE
    sy = (np.arange(H, dtype=np.float64) + 0.5) * STRIDE
    yy, xx = np.meshgrid(sy, sx, indexing='ij')
    shifts = np.stack([xx.ravel(), yy.ravel(), xx.ravel(), yy.ravel()], axis=1)
    anchors = (shifts[:, None, :] + base[None, :, :]).reshape(-1, 4)
    return jnp.asarray(anchors, dtype=jnp.float32)


_ANCHORS = _make_anchors()


CH = 8
NCH = BLK // CH


def _nms_kernel(boxes8_ref, bt_ref, keep_ref):
    bt = bt_ref[0]
    x0r = bt[0:1]
    y0r = bt[1:2]
    x1r = bt[2:3]
    y1r = bt[3:4]
    area_r = (x1r - x0r) * (y1r - y0r)

    col128 = jax.lax.broadcasted_iota(jnp.int32, (1, BLK), 1)
    colg = jax.lax.broadcasted_iota(jnp.int32, (1, NPAD), 1)
    sub8 = jax.lax.broadcasted_iota(jnp.int32, (CH, 1), 0)
    keep = jnp.ones((1, NPAD), dtype=jnp.float32)

    for b in range(NBLK):
        lo = b * BLK
        wid = NPAD - lo
        kb0 = keep[:, lo:lo + BLK]
        acc0 = jnp.zeros((1, wid), dtype=jnp.float32)
        x0t = x0r[:, lo:]
        y0t = y0r[:, lo:]
        x1t = x1r[:, lo:]
        y1t = y1r[:, lo:]
        area_t = area_r[:, lo:]

        def chunk_body(k, carry, b=b, lo=lo):
            kb, acc = carry
            ch = boxes8_ref[0, b * NCH + k]
            x0c = ch[:, 0:1]
            y0c = ch[:, 1:2]
            x1c = ch[:, 2:3]
            y1c = ch[:, 3:4]
            area_c = (x1c - x0c) * (y1c - y0c)
            iw = jnp.maximum(jnp.minimum(x1c, x1t) - jnp.maximum(x0c, x0t), 0.0)
            ih = jnp.maximum(jnp.minimum(y1c, y1t) - jnp.maximum(y0c, y0t), 0.0)
            inter = iw * ih
            union = area_c + area_t - inter
            iou = inter / jnp.maximum(union, 1e-9)
            sup8 = jnp.where(iou > NMS_THRESH, 1.0, 0.0)
            kcol = jnp.zeros((CH, 1), dtype=jnp.float32)
            for j in range(CH):
                ii = k * CH + j
                k_i = jnp.sum(jnp.where(col128 == ii, kb, 0.0))
                row = sup8[j:j + 1, 0:BLK]
                hit = jnp.where((col128 > ii) & (row > 0.0), 1.0, 0.0)
                kb = kb * (1.0 - k_i * hit)
                kcol = jnp.where(sub8 == j, k_i, kcol)
            acc = jnp.maximum(acc, jnp.max(sup8 * kcol, axis=0, keepdims=True))
            return kb, acc

        kb, acc = jax.lax.fori_loop(0, NCH, chunk_body, (kb0, acc0))
        pieces = []
        if lo > 0:
            pieces.append(keep[:, :lo])
        pieces.append(kb)
        if lo + BLK < NPAD:
            pieces.append(keep[:, lo + BLK:])
        keep = jnp.concatenate(pieces, axis=1) if len(pieces) > 1 else kb
        colt = jax.lax.broadcasted_iota(jnp.int32, (1, wid), 1)
        accg = jnp.where((colt >= BLK) & (acc > 0.0), 0.0, 1.0)
        if lo > 0:
            accg = jnp.concatenate(
                [jnp.ones((1, lo), dtype=jnp.float32), accg], axis=1)
        keep = keep * accg

    keep_ref[0] = keep


def _run_nms(top_boxes):
    pad = jnp.zeros((B, NPAD - PRE_NMS, 4), dtype=jnp.float32)
    bx = jnp.concatenate([top_boxes, pad], axis=1)
    bt = jnp.transpose(bx, (0, 2, 1))
    bx8 = bx.reshape(B, NPAD // CH, CH, 4)
    keep = pl.pallas_call(
        _nms_kernel,
        out_shape=jax.ShapeDtypeStruct((B, 1, NPAD), jnp.float32),
        grid=(B,),
        in_specs=[
            pl.BlockSpec((1, NPAD // CH, CH, 4), lambda b: (b, 0, 0, 0)),
            pl.BlockSpec((1, 4, NPAD), lambda b: (b, 0, 0)),
        ],
        out_specs=pl.BlockSpec((1, 1, NPAD), lambda b: (b, 0, 0)),
    )(bx8, bt)
    return keep[:, 0, :PRE_NMS]


def _conv2d(x, w, b, pad):
    out = jax.lax.conv_general_dilated(
        x, w, (1, 1), [(pad, pad), (pad, pad)],
        dimension_numbers=('NCHW', 'OIHW', 'NCHW'))
    return out + b[None, :, None, None]


def _decode(deltas, anchors):
    wa = anchors[:, 2] - anchors[:, 0]
    ha = anchors[:, 3] - anchors[:, 1]
    cxa = anchors[:, 0] + 0.5 * wa
    cya = anchors[:, 1] + 0.5 * ha
    dx = deltas[..., 0]
    dy = deltas[..., 1]
    dw = jnp.minimum(deltas[..., 2], BBOX_CLIP)
    dh = jnp.minimum(deltas[..., 3], BBOX_CLIP)
    cx = dx * wa + cxa
    cy = dy * ha + cya
    w = wa * jnp.exp(dw)
    h = ha * jnp.exp(dh)
    boxes = jnp.stack([cx - 0.5 * w, cy - 0.5 * h, cx + 0.5 * w, cy + 0.5 * h],
                      axis=-1)
    return jnp.clip(boxes, 0.0, IMG)


def kernel(features, conv_w, conv_b, cls_w, cls_b, bbox_w, bbox_b):
    t = jax.nn.relu(_conv2d(features, conv_w, conv_b, 1))
    logits = _conv2d(t, cls_w, cls_b, 0)
    deltas = _conv2d(t, bbox_w, bbox_b, 0)
    logits = jnp.transpose(logits, (0, 2, 3, 1)).reshape(B, -1)
    deltas = deltas.reshape(B, A, 4, H, W)
    deltas = jnp.transpose(deltas, (0, 3, 4, 1, 2)).reshape(B, -1, 4)
    scores = jax.nn.sigmoid(logits)
    boxes = _decode(deltas, _ANCHORS)

    top_scores, idx = jax.lax.top_k(scores, PRE_NMS)
    top_boxes = jnp.take_along_axis(boxes, idx[..., None], axis=1)

    keep = _run_nms(top_boxes)

    sel = jnp.where(keep > 0.0, top_scores, -jnp.inf)
    out_scores, oidx = jax.lax.top_k(sel, POST_NMS)
    out_boxes = jnp.take_along_axis(top_boxes, oidx[..., None], axis=1)
    out_scores = jnp.where(jnp.isfinite(out_scores), out_scores, 0.0)
    return jnp.concatenate([out_boxes, out_scores[..., None]], axis=-1)
